# initial kernel scaffold (unmeasured)
import jax
import jax.numpy as jnp
from jax import lax
from jax.experimental import pallas as pl
from jax.experimental.pallas import tpu as pltpu

N_DEV = 4
M_PER = 2048
K = 8192
HALF = M_PER // 2


def _ag_body(x_ref, xg_ref, copy_sem, send_sems, recv_sems):
    my = lax.axis_index("i")
    left = (my - 1) % N_DEV
    right = (my + 1) % N_DEV

    barrier_sem = pltpu.get_barrier_semaphore()
    for nbr in (left, right):
        pl.semaphore_signal(
            barrier_sem, inc=1,
            device_id=(nbr,), device_id_type=pl.DeviceIdType.MESH,
        )
    pl.semaphore_wait(barrier_sem, 2)

    local = pltpu.make_async_copy(x_ref, xg_ref.at[my], copy_sem)
    local.start()

    full_r = pltpu.make_async_remote_copy(
        src_ref=x_ref, dst_ref=xg_ref.at[my],
        send_sem=send_sems.at[0], recv_sem=recv_sems.at[0],
        device_id=(right,), device_id_type=pl.DeviceIdType.MESH,
    )
    full_l = pltpu.make_async_remote_copy(
        src_ref=x_ref, dst_ref=xg_ref.at[my],
        send_sem=send_sems.at[1], recv_sem=recv_sems.at[1],
        device_id=(left,), device_id_type=pl.DeviceIdType.MESH,
    )
    full_r.start()
    full_l.start()

    full_r.wait_recv()
    half_r = pltpu.make_async_remote_copy(
        src_ref=xg_ref.at[left, pl.ds(0, HALF)],
        dst_ref=xg_ref.at[left, pl.ds(0, HALF)],
        send_sem=send_sems.at[2], recv_sem=recv_sems.at[2],
        device_id=(right,), device_id_type=pl.DeviceIdType.MESH,
    )
    half_r.start()

    full_l.wait_recv()
    half_l = pltpu.make_async_remote_copy(
        src_ref=xg_ref.at[right, pl.ds(HALF, HALF)],
        dst_ref=xg_ref.at[right, pl.ds(HALF, HALF)],
        send_sem=send_sems.at[3], recv_sem=recv_sems.at[3],
        device_id=(left,), device_id_type=pl.DeviceIdType.MESH,
    )
    half_l.start()

    half_r.wait_recv()
    half_l.wait_recv()
    full_r.wait_send()
    full_l.wait_send()
    half_r.wait_send()
    half_l.wait_send()
    local.wait()


def _all_gather(x):
    return pl.pallas_call(
        _ag_body,
        out_shape=jax.ShapeDtypeStruct((N_DEV, M_PER, K), x.dtype),
        in_specs=[pl.BlockSpec(memory_space=pltpu.ANY)],
        out_specs=pl.BlockSpec(memory_space=pltpu.ANY),
        scratch_shapes=[
            pltpu.SemaphoreType.DMA,
            pltpu.SemaphoreType.DMA((4,)),
            pltpu.SemaphoreType.DMA((4,)),
        ],
        compiler_params=pltpu.CompilerParams(collective_id=0),
    )(x)


BM = 512


def _gemm_body(x_ref, w_ref, out_ref):
    out_ref[...] = jnp.dot(
        x_ref[...], w_ref[...], preferred_element_type=jnp.float32
    )


def _gemm(xg, w):
    m = N_DEV * M_PER
    n = w.shape[1]
    return pl.pallas_call(
        _gemm_body,
        grid=(m // BM,),
        in_specs=[
            pl.BlockSpec((BM, K), lambda i: (i, 0)),
            pl.BlockSpec((K, n), lambda i: (0, 0)),
        ],
        out_specs=pl.BlockSpec((BM, n), lambda i: (i, 0)),
        out_shape=jax.ShapeDtypeStruct((m, n), jnp.float32),
        compiler_params=pltpu.CompilerParams(
            dimension_semantics=("arbitrary",),
        ),
    )(xg, w)


def kernel(x, w_mat):
    xg = _all_gather(x)
    xg = xg.reshape(N_DEV * M_PER, K)
    return _gemm(xg, w_mat)


# baseline (device time: 1217652 ns/iter reference)
import jax
import jax.numpy as jnp
from jax import lax
from jax.experimental import pallas as pl
from jax.experimental.pallas import tpu as pltpu

N_DEV = 4
M_PER = 2048
K = 8192
HALF = M_PER // 2


def _ag_body(x_ref, xg_ref, copy_sem, send_sems, recv_sems):
    my = lax.axis_index("i")
    left = (my - 1) % N_DEV
    right = (my + 1) % N_DEV

    barrier_sem = pltpu.get_barrier_semaphore()
    for nbr in (left, right):
        pl.semaphore_signal(
            barrier_sem, inc=1,
            device_id=(nbr,), device_id_type=pl.DeviceIdType.MESH,
        )
    pl.semaphore_wait(barrier_sem, 2)

    local = pltpu.make_async_copy(x_ref, xg_ref.at[my], copy_sem)
    local.start()

    full_r = pltpu.make_async_remote_copy(
        src_ref=x_ref, dst_ref=xg_ref.at[my],
        send_sem=send_sems.at[0], recv_sem=recv_sems.at[0],
        device_id=(right,), device_id_type=pl.DeviceIdType.MESH,
    )
    full_l = pltpu.make_async_remote_copy(
        src_ref=x_ref, dst_ref=xg_ref.at[my],
        send_sem=send_sems.at[1], recv_sem=recv_sems.at[1],
        device_id=(left,), device_id_type=pl.DeviceIdType.MESH,
    )
    full_r.start()
    full_l.start()

    full_r.wait_recv()
    half_r = pltpu.make_async_remote_copy(
        src_ref=xg_ref.at[left, pl.ds(0, HALF)],
        dst_ref=xg_ref.at[left, pl.ds(0, HALF)],
        send_sem=send_sems.at[2], recv_sem=recv_sems.at[2],
        device_id=(right,), device_id_type=pl.DeviceIdType.MESH,
    )
    half_r.start()

    full_l.wait_recv()
    half_l = pltpu.make_async_remote_copy(
        src_ref=xg_ref.at[right, pl.ds(HALF, HALF)],
        dst_ref=xg_ref.at[right, pl.ds(HALF, HALF)],
        send_sem=send_sems.at[3], recv_sem=recv_sems.at[3],
        device_id=(left,), device_id_type=pl.DeviceIdType.MESH,
    )
    half_l.start()

    half_r.wait_recv()
    half_l.wait_recv()
    full_r.wait_send()
    full_l.wait_send()
    half_r.wait_send()
    half_l.wait_send()
    local.wait()


def _all_gather(x):
    return pl.pallas_call(
        _ag_body,
        out_shape=jax.ShapeDtypeStruct((N_DEV, M_PER, K), x.dtype),
        in_specs=[pl.BlockSpec(memory_space=pl.ANY)],
        out_specs=pl.BlockSpec(memory_space=pl.ANY),
        scratch_shapes=[
            pltpu.SemaphoreType.DMA,
            pltpu.SemaphoreType.DMA((4,)),
            pltpu.SemaphoreType.DMA((4,)),
        ],
        compiler_params=pltpu.CompilerParams(collective_id=0),
    )(x)


BM = 512


def _gemm_body(x_ref, w_ref, out_ref):
    out_ref[...] = jnp.dot(
        x_ref[...], w_ref[...], preferred_element_type=jnp.float32
    )


def _gemm(xg, w):
    m = N_DEV * M_PER
    n = w.shape[1]
    return pl.pallas_call(
        _gemm_body,
        grid=(m // BM,),
        in_specs=[
            pl.BlockSpec((BM, K), lambda i: (i, 0)),
            pl.BlockSpec((K, n), lambda i: (0, 0)),
        ],
        out_specs=pl.BlockSpec((BM, n), lambda i: (i, 0)),
        out_shape=jax.ShapeDtypeStruct((m, n), jnp.float32),
        compiler_params=pltpu.CompilerParams(
            dimension_semantics=("arbitrary",),
        ),
    )(xg, w)


def kernel(x, w_mat):
    x = x.astype(jnp.bfloat16)
    w_mat = w_mat.astype(jnp.bfloat16)
    xg = _all_gather(x)
    xg = xg.reshape(N_DEV * M_PER, K)
    return _gemm(xg, w_mat)


# device time: 729891 ns/iter; 1.6683x vs baseline; 1.6683x over previous
import jax
import jax.numpy as jnp
from jax import lax
from jax.experimental import pallas as pl
from jax.experimental.pallas import tpu as pltpu

N_DEV = 4
M_PER = 2048
K = 8192
HALF = M_PER // 2
BM = 512


def _body(x_ref, w_ref, out_ref, xg_ref, xtile, otile, copy_sem,
          send_sems, recv_sems):
    my = lax.axis_index("i")
    left = (my - 1) % N_DEV
    right = (my + 1) % N_DEV

    barrier_sem = pltpu.get_barrier_semaphore()
    for nbr in (left, right):
        pl.semaphore_signal(
            barrier_sem, inc=1,
            device_id=(nbr,), device_id_type=pl.DeviceIdType.MESH,
        )
    pl.semaphore_wait(barrier_sem, 2)

    full_r = pltpu.make_async_remote_copy(
        src_ref=x_ref, dst_ref=xg_ref.at[my],
        send_sem=send_sems.at[0], recv_sem=recv_sems.at[0],
        device_id=(right,), device_id_type=pl.DeviceIdType.MESH,
    )
    full_l = pltpu.make_async_remote_copy(
        src_ref=x_ref, dst_ref=xg_ref.at[my],
        send_sem=send_sems.at[1], recv_sem=recv_sems.at[1],
        device_id=(left,), device_id_type=pl.DeviceIdType.MESH,
    )
    full_r.start()
    full_l.start()

    def gemm_chunk(src_ref, origin):
        def step(t, carry):
            ld = pltpu.make_async_copy(
                src_ref.at[pl.ds(t * BM, BM)], xtile, copy_sem)
            ld.start()
            ld.wait()
            otile[...] = jnp.dot(
                xtile[...], w_ref[...], preferred_element_type=jnp.float32)
            st = pltpu.make_async_copy(
                otile, out_ref.at[pl.ds(origin * M_PER + t * BM, BM)],
                copy_sem)
            st.start()
            st.wait()
            return carry

        lax.fori_loop(0, M_PER // BM, step, 0)

    gemm_chunk(x_ref, my)

    full_r.wait_recv()
    half_r = pltpu.make_async_remote_copy(
        src_ref=xg_ref.at[left, pl.ds(0, HALF)],
        dst_ref=xg_ref.at[left, pl.ds(0, HALF)],
        send_sem=send_sems.at[2], recv_sem=recv_sems.at[2],
        device_id=(right,), device_id_type=pl.DeviceIdType.MESH,
    )
    half_r.start()
    gemm_chunk(xg_ref.at[left], left)

    full_l.wait_recv()
    half_l = pltpu.make_async_remote_copy(
        src_ref=xg_ref.at[right, pl.ds(HALF, HALF)],
        dst_ref=xg_ref.at[right, pl.ds(HALF, HALF)],
        send_sem=send_sems.at[3], recv_sem=recv_sems.at[3],
        device_id=(left,), device_id_type=pl.DeviceIdType.MESH,
    )
    half_l.start()
    gemm_chunk(xg_ref.at[right], right)

    half_r.wait_recv()
    half_l.wait_recv()
    diag = (my + 2) % N_DEV
    gemm_chunk(xg_ref.at[diag], diag)

    full_r.wait_send()
    full_l.wait_send()
    half_r.wait_send()
    half_l.wait_send()


def kernel(x, w_mat):
    x = x.astype(jnp.bfloat16)
    w_mat = w_mat.astype(jnp.bfloat16)
    n = w_mat.shape[1]
    out, _xg = pl.pallas_call(
        _body,
        out_shape=[
            jax.ShapeDtypeStruct((N_DEV * M_PER, n), jnp.float32),
            jax.ShapeDtypeStruct((N_DEV, M_PER, K), jnp.bfloat16),
        ],
        in_specs=[
            pl.BlockSpec(memory_space=pl.ANY),
            pl.BlockSpec(memory_space=pltpu.VMEM),
        ],
        out_specs=[
            pl.BlockSpec(memory_space=pl.ANY),
            pl.BlockSpec(memory_space=pl.ANY),
        ],
        scratch_shapes=[
            pltpu.VMEM((BM, K), jnp.bfloat16),
            pltpu.VMEM((BM, n), jnp.float32),
            pltpu.SemaphoreType.DMA,
            pltpu.SemaphoreType.DMA((4,)),
            pltpu.SemaphoreType.DMA((4,)),
        ],
        compiler_params=pltpu.CompilerParams(collective_id=0),
    )(x, w_mat)
    return out


# device time: 606700 ns/iter; 2.0070x vs baseline; 1.2031x over previous
import jax
import jax.numpy as jnp
from jax import lax
from jax.experimental import pallas as pl
from jax.experimental.pallas import tpu as pltpu

N_DEV = 4
M_PER = 2048
K = 8192
N_OUT = 1024
BM = 512
NB = M_PER // BM


def _body(x_ref, w_ref, out_ref, xg_ref,
          xf32, xtile, w_bf, wf32, otile,
          copy_sem, send_r, send_l, recv_l, recv_r):
    my = lax.axis_index("i")
    left = (my - 1) % N_DEV
    right = (my + 1) % N_DEV
    diag = (my + 2) % N_DEV

    def blk(chunk, b):
        return xg_ref.at[chunk, pl.ds(b * BM, BM)]

    def rdma(src, dst, ssem, rsem, dev):
        return pltpu.make_async_remote_copy(
            src_ref=src, dst_ref=dst, send_sem=ssem, recv_sem=rsem,
            device_id=(dev,), device_id_type=pl.DeviceIdType.MESH)

    barrier_sem = pltpu.get_barrier_semaphore()
    for nbr in (left, right):
        pl.semaphore_signal(
            barrier_sem, inc=1,
            device_id=(nbr,), device_id_type=pl.DeviceIdType.MESH)
    pl.semaphore_wait(barrier_sem, 2)

    def conv_send(b, c):
        ld = pltpu.make_async_copy(x_ref.at[pl.ds(b * BM, BM)], xf32, copy_sem)
        ld.start()
        ld.wait()
        xtile[...] = xf32[...].astype(jnp.bfloat16)
        st = pltpu.make_async_copy(xtile, blk(my, b), copy_sem)
        st.start()
        st.wait()
        rdma(blk(my, b), blk(my, b), send_r.at[b], recv_l.at[b], right).start()
        rdma(blk(my, b), blk(my, b), send_l.at[b], recv_r.at[b], left).start()
        return c

    lax.fori_loop(0, NB, conv_send, 0)

    def conv_w(j, c):
        ld = pltpu.make_async_copy(
            w_ref.at[pl.ds(j * 2048, 2048)], wf32, copy_sem)
        ld.start()
        ld.wait()
        w_bf[pl.ds(j * 2048, 2048), :] = wf32[...].astype(jnp.bfloat16)
        return c

    lax.fori_loop(0, K // 2048, conv_w, 0)

    def gemm(chunk, b):
        ld = pltpu.make_async_copy(blk(chunk, b), xtile, copy_sem)
        ld.start()
        ld.wait()
        otile[...] = jnp.dot(
            xtile[...], w_bf[...], preferred_element_type=jnp.float32)
        st = pltpu.make_async_copy(
            otile, out_ref.at[pl.ds(chunk * M_PER + b * BM, BM)], copy_sem)
        st.start()
        st.wait()

    lax.fori_loop(0, NB, lambda b, c: (gemm(my, b), c)[1], 0)

    def left_right(b, c):
        rdma(blk(left, b), blk(left, b),
             send_r.at[b], recv_l.at[b], right).wait_recv()

        @pl.when(b < 2)
        def _():
            rdma(blk(left, b), blk(left, b),
                 send_r.at[4 + b], recv_l.at[4 + b], right).start()

        gemm(left, b)

        rdma(blk(right, b), blk(right, b),
             send_l.at[b], recv_r.at[b], left).wait_recv()

        @pl.when(b >= 2)
        def _():
            rdma(blk(right, b), blk(right, b),
                 send_l.at[2 + b], recv_r.at[2 + b], left).start()

        gemm(right, b)
        return c

    lax.fori_loop(0, NB, left_right, 0)

    def diag_blocks(j, c):
        rdma(blk(diag, j), blk(diag, j),
             send_r.at[4 + j], recv_l.at[4 + j], right).wait_recv()
        gemm(diag, j)
        rdma(blk(diag, 2 + j), blk(diag, 2 + j),
             send_l.at[4 + j], recv_r.at[4 + j], left).wait_recv()
        gemm(diag, 2 + j)
        return c

    lax.fori_loop(0, 2, diag_blocks, 0)

    def drain(i, c):
        rdma(blk(0, 0), blk(0, 0),
             send_r.at[i], recv_l.at[i], right).wait_send()
        rdma(blk(0, 0), blk(0, 0),
             send_l.at[i], recv_r.at[i], left).wait_send()
        return c

    lax.fori_loop(0, 6, drain, 0)


def kernel(x, w_mat):
    out, _xg = pl.pallas_call(
        _body,
        out_shape=[
            jax.ShapeDtypeStruct((N_DEV * M_PER, N_OUT), jnp.float32),
            jax.ShapeDtypeStruct((N_DEV, M_PER, K), jnp.bfloat16),
        ],
        in_specs=[
            pl.BlockSpec(memory_space=pl.ANY),
            pl.BlockSpec(memory_space=pl.ANY),
        ],
        out_specs=[
            pl.BlockSpec(memory_space=pl.ANY),
            pl.BlockSpec(memory_space=pl.ANY),
        ],
        scratch_shapes=[
            pltpu.VMEM((BM, K), jnp.float32),
            pltpu.VMEM((BM, K), jnp.bfloat16),
            pltpu.VMEM((K, N_OUT), jnp.bfloat16),
            pltpu.VMEM((2048, N_OUT), jnp.float32),
            pltpu.VMEM((BM, N_OUT), jnp.float32),
            pltpu.SemaphoreType.DMA,
            pltpu.SemaphoreType.DMA((6,)),
            pltpu.SemaphoreType.DMA((6,)),
            pltpu.SemaphoreType.DMA((6,)),
            pltpu.SemaphoreType.DMA((6,)),
        ],
        compiler_params=pltpu.CompilerParams(
            collective_id=0,
            vmem_limit_bytes=58 * 1024 * 1024,
        ),
    )(x, w_mat)
    return out
